# baseline (speedup 1.0000x reference)
import functools

import jax
import jax.numpy as jnp
from jax import lax
from jax.experimental import pallas as pl
from jax.experimental.pallas import tpu as pltpu
from jax.experimental.pallas import tpu_sc as plsc

EPS = 1e-4
NTOK = 32768
TH = 2048
KTOP = 2048
NV = NTOK // 16
CMAX = 4096
CLIM = CMAX + 16
CBUF = CMAX + 32
SELB = 1024
SORTB = 256


def _log_priors(U_t, U_h):
    b = U_t.shape[0]
    t, h = U_t.shape[1], U_h.shape[1]
    ut_e = jnp.broadcast_to(U_t[:, :, None], (b, t, h)).reshape(b, TH)
    uh_e = jnp.broadcast_to(U_h[:, None, :], (b, t, h)).reshape(b, TH)
    fs = jnp.exp(jnp.log(ut_e) / 1.0) * jnp.exp(jnp.log(uh_e) / 2.0)
    fs = jnp.clip(fs, EPS, 1.0 - EPS)
    ft = jnp.exp(jnp.log(1.0 - ut_e) / 1.0) * jnp.exp(jnp.log(1.0 - uh_e) / 2.0)
    ft = jnp.clip(ft, EPS, 1.0 - EPS)
    return jnp.log(fs), jnp.log(ft)


def _sc_body(gs_hbm, gt_hbm, ps_hbm, pt_hbm, os_hbm, ot_hbm,
             gbuf, kbuf, pbuf, hist, tot, cka, cia, ckb, cib, smem):
    cid = lax.axis_index("c")
    sid = lax.axis_index("s")
    wid = sid * 2 + cid
    pair = wid % 2
    base = wid // 2
    lane = lax.iota(jnp.int32, 16)
    ones16 = jnp.full((16,), -1, jnp.int32)

    def zero_hist(nwords):
        def z(i, _):
            hist[pl.ds(i * 16, 16)] = jnp.zeros((16,), jnp.int32)
            return 0
        lax.fori_loop(0, nwords // 16, z, 0)

    def bin_totals_cum(nbins):
        def tots(t, carry):
            acc = hist[pl.ds(t * 16, 16)]
            for l in range(1, 16):
                acc = acc + hist[pl.ds(l * SELB + t * 16, 16)]
            cst = plsc.cumsum(acc) + carry
            tot[pl.ds(t * 16, 16)] = cst
            return jnp.max(cst)
        lax.fori_loop(0, nbins // 16, tots, jnp.int32(0))

    def find_cut(nbins, kneed):
        def findd(t, acc):
            v = tot[pl.ds(t * 16, 16)]
            return acc + jnp.sum(jnp.where(v < kneed, 1, 0))
        return lax.fori_loop(0, nbins // 16, findd, jnp.int32(0))

    def process(g_hbm, p_hbm, o_hbm):
        def task(j, _):
            b = base + 16 * j
            pltpu.sync_copy(g_hbm.at[b], gbuf)
            pltpu.sync_copy(p_hbm.at[b], pbuf.at[pl.ds(0, TH)])

            zero_hist(SELB * 16)

            def passa(jv, _c):
                g = gbuf[pl.ds(jv * 16, 16)]
                pv = pbuf[pl.ds(jv, 16)][0]
                s = g + pv
                bits = lax.bitcast_convert_type(s, jnp.uint32)
                neg = bits >= jnp.uint32(0x80000000)
                m = jnp.where(neg, ~bits, bits | jnp.uint32(0x80000000))
                key = ~m
                kbuf[pl.ds(jv * 16, 16)] = key
                d = lax.bitcast_convert_type(key >> 22, jnp.int32)
                hidx = lane * SELB + d
                c = plsc.load_gather(hist, [hidx])
                plsc.store_scatter(hist, [hidx], c + 1)
                return 0
            lax.fori_loop(0, NV, passa, 0)

            bin_totals_cum(SELB)
            d1 = find_cut(SELB, KTOP)
            cm1 = tot[pl.ds(jnp.maximum(d1 - 1, 0), 16)][0]
            smem[0] = d1
            smem[1] = 22
            smem[2] = jnp.where(d1 > 0, cm1, 0)
            smem[3] = tot[pl.ds(d1, 16)][0]

            for (s_prev, s_cur, wbits) in ((22, 12, 10), (12, 2, 10), (2, 0, 2)):
                nb = 1 << wbits
                nbt = max(nb, 16)

                @pl.when(smem[3] > CMAX)
                def _():
                    p_prev = lax.bitcast_convert_type(
                        jnp.full((16,), smem[0], jnp.int32), jnp.uint32)
                    cl0 = smem[2]
                    zero_hist(nbt * 16)

                    def rpass(jv, _c):
                        key = kbuf[pl.ds(jv * 16, 16)]
                        mk = (key >> s_prev) == p_prev
                        d = lax.bitcast_convert_type(
                            (key >> s_cur) & jnp.uint32(nb - 1), jnp.int32)
                        hidx = lane * SELB + d
                        c = plsc.load_gather(hist, [hidx])
                        plsc.store_scatter(hist, [hidx], c + 1, mask=mk)
                        return 0
                    lax.fori_loop(0, NV, rpass, 0)

                    bin_totals_cum(nbt)
                    d2 = find_cut(nbt, KTOP - cl0)
                    cm2 = tot[pl.ds(jnp.maximum(d2 - 1, 0), 16)][0]
                    smem[3] = cl0 + tot[pl.ds(d2, 16)][0]
                    smem[2] = cl0 + jnp.where(d2 > 0, cm2, 0)
                    smem[0] = smem[0] * nb + d2
                    smem[1] = s_cur

            sfin = jnp.full((16,), smem[1], jnp.int32).astype(jnp.uint32)
            pfin = lax.bitcast_convert_type(jnp.full((16,), smem[0], jnp.int32), jnp.uint32)

            def extr(jv, off):
                key = kbuf[pl.ds(jv * 16, 16)]
                mk = (key >> sfin) <= pfin
                inc = plsc.cumsum(jnp.where(mk, 1, 0))
                pos = off + inc - 1
                stm = mk & (pos < CLIM)
                plsc.store_scatter(cka, [pos], lax.bitcast_convert_type(key, jnp.int32),
                                   mask=stm)
                plsc.store_scatter(cia, [pos], jv * 16 + lane, mask=stm)
                return off + jnp.max(inc)
            off = lax.fori_loop(0, NV, extr, jnp.int32(0))
            m_cnt = jnp.minimum(off, CLIM)
            m_pad = (m_cnt + 15) & ~15

            plsc.store_scatter(cka, [m_cnt + lane], ones16,
                               mask=lane < (m_pad - m_cnt))

            q = m_pad // 16
            bidx = lane * q
            for p in range(4):
                s_k, s_i, d_k, d_i = ((cka, cia, ckb, cib) if p % 2 == 0
                                      else (ckb, cib, cka, cia))
                sh = 8 * p

                def zs(i, _c):
                    hist[pl.ds(i * 16, 16)] = jnp.zeros((16,), jnp.int32)
                    return 0
                lax.fori_loop(0, SORTB, zs, 0)

                def hpass(i, _c, s_k=s_k, sh=sh):
                    k = plsc.load_gather(s_k, [bidx + i])
                    d = lax.shift_right_logical(k, sh) & 0xFF
                    hidx = d * 16 + lane
                    c = plsc.load_gather(hist, [hidx])
                    plsc.store_scatter(hist, [hidx], c + 1)
                    return 0
                lax.fori_loop(0, q, hpass, 0)

                def scan(i, carry):
                    v = hist[pl.ds(i * 16, 16)]
                    cs = plsc.cumsum(v)
                    hist[pl.ds(i * 16, 16)] = cs - v + carry
                    return carry + jnp.max(cs)
                lax.fori_loop(0, SORTB, scan, jnp.int32(0))

                def ppass(i, _c, s_k=s_k, s_i=s_i, d_k=d_k, d_i=d_i, sh=sh):
                    ii = bidx + i
                    k = plsc.load_gather(s_k, [ii])
                    v = plsc.load_gather(s_i, [ii])
                    d = lax.shift_right_logical(k, sh) & 0xFF
                    hidx = d * 16 + lane
                    r = plsc.load_gather(hist, [hidx])
                    plsc.store_scatter(hist, [hidx], r + 1)
                    plsc.store_scatter(d_k, [r], k)
                    plsc.store_scatter(d_i, [r], v)
                    return 0
                lax.fori_loop(0, q, ppass, 0)

            pltpu.sync_copy(cia.at[pl.ds(0, KTOP)], o_hbm.at[b])
            return 0

        lax.fori_loop(0, 64, task, 0)

    @pl.when(pair == 0)
    def _():
        process(gs_hbm, ps_hbm, os_hbm)

    @pl.when(pair == 1)
    def _():
        process(gt_hbm, pt_hbm, ot_hbm)


def kernel(U_t, U_h, G_src, G_tgt, B):
    b = U_t.shape[0]
    ls, lt = _log_priors(U_t, U_h)

    mesh = plsc.VectorSubcoreMesh(core_axis_name="c", subcore_axis_name="s")
    sc = pl.kernel(
        _sc_body,
        mesh=mesh,
        compiler_params=pltpu.CompilerParams(needs_layout_passes=False),
        out_type=(jax.ShapeDtypeStruct((b, KTOP), jnp.int32),
                  jax.ShapeDtypeStruct((b, KTOP), jnp.int32)),
        scratch_types=[
            pltpu.VMEM((NTOK,), jnp.float32),
            pltpu.VMEM((NTOK,), jnp.uint32),
            pltpu.VMEM((TH + 16,), jnp.float32),
            pltpu.VMEM((SELB * 16,), jnp.int32),
            pltpu.VMEM((SELB + 16,), jnp.int32),
            pltpu.VMEM((CBUF,), jnp.int32),
            pltpu.VMEM((CBUF,), jnp.int32),
            pltpu.VMEM((CBUF,), jnp.int32),
            pltpu.VMEM((CBUF,), jnp.int32),
            pltpu.SMEM((8,), jnp.int32),
        ],
    )
    src_idx, tgt_idx = sc(G_src, G_tgt, ls, lt)
    return (src_idx, tgt_idx)


# speedup vs baseline: 1.0605x; 1.0605x over previous
import jax
import jax.numpy as jnp
from jax import lax
from jax.experimental import pallas as pl
from jax.experimental.pallas import tpu as pltpu
from jax.experimental.pallas import tpu_sc as plsc

EPS = 1e-4
NTOK = 32768
TH = 2048
KTOP = 2048
NG = TH // 16
CMAX = 8192
CLIM = CMAX + 16
CBUF = CMAX + 80
SELB = 1024
SORTB = 256
NROW = 64


def _log_priors(U_t, U_h):
    b = U_t.shape[0]
    t, h = U_t.shape[1], U_h.shape[1]
    ut_e = jnp.broadcast_to(U_t[:, :, None], (b, t, h)).reshape(b, TH)
    uh_e = jnp.broadcast_to(U_h[:, None, :], (b, t, h)).reshape(b, TH)
    fs = jnp.exp(jnp.log(ut_e) / 1.0) * jnp.exp(jnp.log(uh_e) / 2.0)
    fs = jnp.clip(fs, EPS, 1.0 - EPS)
    ft = jnp.exp(jnp.log(1.0 - ut_e) / 1.0) * jnp.exp(jnp.log(1.0 - uh_e) / 2.0)
    ft = jnp.clip(ft, EPS, 1.0 - EPS)
    return jnp.log(fs), jnp.log(ft)


def _mkkey(g, pv):
    s = g + pv
    bits = lax.bitcast_convert_type(s, jnp.uint32)
    neg = bits >= jnp.uint32(0x80000000)
    m = jnp.where(neg, ~bits, bits | jnp.uint32(0x80000000))
    return ~m


def _sc_body(gs_hbm, gt_hbm, ps_hbm, pt_hbm, os_hbm, ot_hbm,
             gbuf0, gbuf1, pbuf0, pbuf1, hist, tot, cka, cia, ckb, cib,
             smem, sg0, sg1, sp0, sp1):
    cid = lax.axis_index("c")
    sid = lax.axis_index("s")
    wid = sid * 2 + cid
    pair = wid % 2
    base = wid // 2
    lane = lax.iota(jnp.int32, 16)
    one16 = jnp.ones((16,), jnp.int32)
    ones16 = jnp.full((16,), -1, jnp.int32)
    lsel = lane * SELB

    def zero_hist(nwords):
        def z(i, _):
            hist[pl.ds(i * 16, 16)] = jnp.zeros((16,), jnp.int32)
            return 0
        lax.fori_loop(0, nwords // 16, z, 0)

    def bin_totals_cum(nbins):
        def tots(t_, carry):
            acc = hist[pl.ds(t_ * 16, 16)]
            for l in range(1, 16):
                acc = acc + hist[pl.ds(l * SELB + t_ * 16, 16)]
            cst = plsc.cumsum(acc) + carry
            tot[pl.ds(t_ * 16, 16)] = cst
            return jnp.max(cst)
        lax.fori_loop(0, nbins // 16, tots, jnp.int32(0))

    def find_cut(nbins, kneed):
        def findd(t_, acc):
            v = tot[pl.ds(t_ * 16, 16)]
            return acc + jnp.sum(jnp.where(v < kneed, 1, 0))
        return lax.fori_loop(0, nbins // 16, findd, jnp.int32(0))

    def process():
        def issue(j, gbuf, pbuf, sg, sp):
            b = base + 16 * j

            @pl.when(pair == 0)
            def _():
                pltpu.make_async_copy(gs_hbm.at[b], gbuf, sg).start()
                pltpu.make_async_copy(ps_hbm.at[b], pbuf.at[pl.ds(0, TH)],
                                      sp).start()

            @pl.when(pair == 1)
            def _():
                pltpu.make_async_copy(gt_hbm.at[b], gbuf, sg).start()
                pltpu.make_async_copy(pt_hbm.at[b], pbuf.at[pl.ds(0, TH)],
                                      sp).start()

        def task(j, gbuf, pbuf, sg, sp):
            b = base + 16 * j
            pltpu.make_async_copy(gs_hbm.at[b], gbuf, sg).wait()
            pltpu.make_async_copy(ps_hbm.at[b], pbuf.at[pl.ds(0, TH)],
                                  sp).wait()

            zero_hist(SELB * 16)

            def agroup(o, _c):
                pvec = pbuf[pl.ds(o * 16, 16)]
                for l in range(16):
                    g = gbuf[pl.ds(o * 256 + l * 16, 16)]
                    key = _mkkey(g, pvec[l])
                    d = lax.bitcast_convert_type(key >> 22, jnp.int32)
                    plsc.addupdate_scatter(hist, [lsel + d], one16)
                return 0
            lax.fori_loop(0, NG, agroup, 0)

            bin_totals_cum(SELB)
            d1 = find_cut(SELB, KTOP)
            cm1 = tot[pl.ds(jnp.maximum(d1 - 1, 0), 16)][0]
            smem[0] = d1
            smem[1] = 22
            smem[2] = jnp.where(d1 > 0, cm1, 0)
            smem[3] = tot[pl.ds(d1, 16)][0]

            for (s_prev, s_cur, wbits) in ((22, 12, 10), (12, 2, 10), (2, 0, 2)):
                nb = 1 << wbits
                nbt = max(nb, 16)

                @pl.when(smem[3] > CMAX)
                def _():
                    p_prev = lax.bitcast_convert_type(
                        jnp.full((16,), smem[0], jnp.int32), jnp.uint32)
                    cl0 = smem[2]
                    zero_hist(nbt * 16)

                    def rgroup(o, _c):
                        pvec = pbuf[pl.ds(o * 16, 16)]
                        for l in range(16):
                            g = gbuf[pl.ds(o * 256 + l * 16, 16)]
                            key = _mkkey(g, pvec[l])
                            mk = (key >> s_prev) == p_prev
                            d = lax.bitcast_convert_type(
                                (key >> s_cur) & jnp.uint32(nb - 1), jnp.int32)
                            plsc.addupdate_scatter(hist, [lsel + d], one16,
                                                   mask=mk)
                        return 0
                    lax.fori_loop(0, NG, rgroup, 0)

                    bin_totals_cum(nbt)
                    d2 = find_cut(nbt, KTOP - cl0)
                    cm2 = tot[pl.ds(jnp.maximum(d2 - 1, 0), 16)][0]
                    smem[3] = cl0 + tot[pl.ds(d2, 16)][0]
                    smem[2] = cl0 + jnp.where(d2 > 0, cm2, 0)
                    smem[0] = smem[0] * nb + d2
                    smem[1] = s_cur

            sfin = jnp.full((16,), smem[1], jnp.int32).astype(jnp.uint32)
            pfin = lax.bitcast_convert_type(
                jnp.full((16,), smem[0], jnp.int32), jnp.uint32)

            def egroup(o, off_vec):
                pvec = pbuf[pl.ds(o * 16, 16)]
                for l in range(16):
                    jv = o * 16 + l
                    g = gbuf[pl.ds(o * 256 + l * 16, 16)]
                    key = _mkkey(g, pvec[l])
                    mk = (key >> sfin) <= pfin
                    inc = plsc.cumsum(jnp.where(mk, 1, 0))
                    pos = off_vec + inc - 1
                    stm = mk & (pos < CLIM)
                    plsc.store_scatter(
                        cka, [pos], lax.bitcast_convert_type(key, jnp.int32),
                        mask=stm)
                    plsc.store_scatter(cia, [pos], jv * 16 + lane, mask=stm)
                    off_vec = off_vec + plsc.all_reduce_population_count(mk)
                return off_vec
            off_vec = lax.fori_loop(0, NG, egroup, jnp.zeros((16,), jnp.int32))
            m_cnt = jnp.minimum(jnp.max(off_vec), CLIM)
            m_pad = (m_cnt + 63) & ~63

            for pi in range(4):
                iv = m_cnt + pi * 16 + lane
                plsc.store_scatter(cka, [iv], ones16, mask=iv < m_pad)

            q = m_pad // 16
            qq = m_pad // 64
            bidx = lane * q
            for p in range(4):
                s_k, s_i, d_k, d_i = ((cka, cia, ckb, cib) if p % 2 == 0
                                      else (ckb, cib, cka, cia))
                sh = 8 * p

                def zs(i, _c):
                    hist[pl.ds(i * 16, 16)] = jnp.zeros((16,), jnp.int32)
                    return 0
                lax.fori_loop(0, SORTB, zs, 0)

                def hpass(i, _c, s_k=s_k, sh=sh):
                    for u in range(4):
                        k = plsc.load_gather(s_k, [bidx + (i * 4 + u)])
                        d = lax.shift_right_logical(k, sh) & 0xFF
                        plsc.addupdate_scatter(hist, [d * 16 + lane], one16)
                    return 0
                lax.fori_loop(0, qq, hpass, 0)

                def scan(i, carry):
                    v = hist[pl.ds(i * 16, 16)]
                    cs = plsc.cumsum(v)
                    hist[pl.ds(i * 16, 16)] = cs - v + carry
                    return carry + jnp.max(cs)
                lax.fori_loop(0, SORTB, scan, jnp.int32(0))

                def ppass(i, _c, s_k=s_k, s_i=s_i, d_k=d_k, d_i=d_i, sh=sh):
                    for u in range(4):
                        ii = bidx + (i * 4 + u)
                        k = plsc.load_gather(s_k, [ii])
                        v = plsc.load_gather(s_i, [ii])
                        d = lax.shift_right_logical(k, sh) & 0xFF
                        hidx = d * 16 + lane
                        r = plsc.load_gather(hist, [hidx])
                        plsc.store_scatter(hist, [hidx], r + 1)
                        plsc.store_scatter(d_k, [r], k)
                        plsc.store_scatter(d_i, [r], v)
                    return 0
                lax.fori_loop(0, qq, ppass, 0)

            @pl.when(pair == 0)
            def _():
                pltpu.sync_copy(cia.at[pl.ds(0, KTOP)], os_hbm.at[b])

            @pl.when(pair == 1)
            def _():
                pltpu.sync_copy(cia.at[pl.ds(0, KTOP)], ot_hbm.at[b])

        issue(0, gbuf0, pbuf0, sg0, sp0)

        def pair_of_tasks(jj, _c):
            j0 = jj * 2
            issue(j0 + 1, gbuf1, pbuf1, sg1, sp1)
            task(j0, gbuf0, pbuf0, sg0, sp0)

            @pl.when(jj < NROW // 2 - 1)
            def _():
                issue(j0 + 2, gbuf0, pbuf0, sg0, sp0)
            task(j0 + 1, gbuf1, pbuf1, sg1, sp1)
            return 0
        lax.fori_loop(0, NROW // 2, pair_of_tasks, 0)

    process()


def kernel(U_t, U_h, G_src, G_tgt, B):
    b = U_t.shape[0]
    ls, lt = _log_priors(U_t, U_h)

    mesh = plsc.VectorSubcoreMesh(core_axis_name="c", subcore_axis_name="s")
    sc = pl.kernel(
        _sc_body,
        mesh=mesh,
        compiler_params=pltpu.CompilerParams(needs_layout_passes=False),
        out_type=(jax.ShapeDtypeStruct((b, KTOP), jnp.int32),
                  jax.ShapeDtypeStruct((b, KTOP), jnp.int32)),
        scratch_types=[
            pltpu.VMEM((NTOK,), jnp.float32),
            pltpu.VMEM((NTOK,), jnp.float32),
            pltpu.VMEM((TH + 16,), jnp.float32),
            pltpu.VMEM((TH + 16,), jnp.float32),
            pltpu.VMEM((SELB * 16,), jnp.int32),
            pltpu.VMEM((SELB + 16,), jnp.int32),
            pltpu.VMEM((CBUF,), jnp.int32),
            pltpu.VMEM((CBUF,), jnp.int32),
            pltpu.VMEM((CBUF,), jnp.int32),
            pltpu.VMEM((CBUF,), jnp.int32),
            pltpu.SMEM((8,), jnp.int32),
            pltpu.SemaphoreType.DMA,
            pltpu.SemaphoreType.DMA,
            pltpu.SemaphoreType.DMA,
            pltpu.SemaphoreType.DMA,
        ],
    )
    src_idx, tgt_idx = sc(G_src, G_tgt, ls, lt)
    return (src_idx, tgt_idx)


# speedup vs baseline: 2.7963x; 2.6368x over previous
import jax
import jax.numpy as jnp
from jax import lax
from jax.experimental import pallas as pl
from jax.experimental.pallas import tpu as pltpu
from jax.experimental.pallas import tpu_sc as plsc

EPS = 1e-4
NTOK = 32768
TH = 2048
KTOP = 2048
NG = TH // 16
CMAX = 8192
CLIM = CMAX + 16
CBUF = CMAX + 80
SELB = 1024
SORTB = 256
NROW = 64


def _log_priors(U_t, U_h):
    b = U_t.shape[0]
    t, h = U_t.shape[1], U_h.shape[1]
    ut_e = jnp.broadcast_to(U_t[:, :, None], (b, t, h)).reshape(b, TH)
    uh_e = jnp.broadcast_to(U_h[:, None, :], (b, t, h)).reshape(b, TH)
    fs = jnp.exp(jnp.log(ut_e) / 1.0) * jnp.exp(jnp.log(uh_e) / 2.0)
    fs = jnp.clip(fs, EPS, 1.0 - EPS)
    ft = jnp.exp(jnp.log(1.0 - ut_e) / 1.0) * jnp.exp(jnp.log(1.0 - uh_e) / 2.0)
    ft = jnp.clip(ft, EPS, 1.0 - EPS)
    return jnp.log(fs), jnp.log(ft)


def _mkkey(g, pv):
    s = g + pv
    bits = lax.bitcast_convert_type(s, jnp.uint32)
    neg = bits >= jnp.uint32(0x80000000)
    m = jnp.where(neg, ~bits, bits | jnp.uint32(0x80000000))
    return ~m


def _sc_body(gs_hbm, gt_hbm, ps_hbm, pt_hbm, os_hbm, ot_hbm,
             gbuf0, gbuf1, pbuf0, pbuf1, hist, tot, cka, cia, ckb, cib,
             smem, sg0, sg1, sp0, sp1):
    cid = lax.axis_index("c")
    sid = lax.axis_index("s")
    wid = sid * 2 + cid
    pair = wid % 2
    base = wid // 2
    lane = lax.iota(jnp.int32, 16)
    one16 = jnp.ones((16,), jnp.int32)
    ones16 = jnp.full((16,), -1, jnp.int32)
    lsel = lane * SELB

    zvec = jnp.zeros((16,), jnp.int32)

    def zero_hist(nwords):
        def z(i, _):
            for u in range(8):
                hist[pl.ds((i * 8 + u) * 16, 16)] = zvec
            return 0
        lax.fori_loop(0, nwords // 128, z, 0)

    def bin_totals_cum(nbins):
        def tots(t_, carry):
            acc = hist[pl.ds(t_ * 16, 16)]
            for l in range(1, 16):
                acc = acc + hist[pl.ds(l * SELB + t_ * 16, 16)]
            cst = plsc.cumsum(acc) + carry
            tot[pl.ds(t_ * 16, 16)] = cst
            return jnp.max(cst)
        lax.fori_loop(0, nbins // 16, tots, jnp.int32(0))

    def find_cut(nbins, kneed):
        def findd(t_, acc):
            v = tot[pl.ds(t_ * 16, 16)]
            return acc + jnp.sum(jnp.where(v < kneed, 1, 0))
        return lax.fori_loop(0, nbins // 16, findd, jnp.int32(0))

    def process():
        def issue(j, gbuf, pbuf, sg, sp):
            b = base + 16 * j

            @pl.when(pair == 0)
            def _():
                pltpu.make_async_copy(gs_hbm.at[b], gbuf, sg).start()
                pltpu.make_async_copy(ps_hbm.at[b], pbuf.at[pl.ds(0, TH)],
                                      sp).start()

            @pl.when(pair == 1)
            def _():
                pltpu.make_async_copy(gt_hbm.at[b], gbuf, sg).start()
                pltpu.make_async_copy(pt_hbm.at[b], pbuf.at[pl.ds(0, TH)],
                                      sp).start()

        def task(j, gbuf, pbuf, sg, sp):
            b = base + 16 * j
            pltpu.make_async_copy(gs_hbm.at[b], gbuf, sg).wait()
            pltpu.make_async_copy(ps_hbm.at[b], pbuf.at[pl.ds(0, TH)],
                                  sp).wait()

            zero_hist(SELB * 16)

            def agroup(o, _c):
                pvec = pbuf[pl.ds(o * 16, 16)]
                for q4 in range(4):
                    ls_ = [o * 256 + (q4 * 4 + u) * 16 for u in range(4)]
                    gs = [gbuf[pl.ds(a, 16)] for a in ls_]
                    ks = [_mkkey(gs[u], pvec[q4 * 4 + u]) for u in range(4)]
                    ds_ = [lax.bitcast_convert_type(k >> 22, jnp.int32)
                           for k in ks]
                    for u in range(4):
                        plsc.addupdate_scatter(hist, [lsel + ds_[u]], one16)
                return 0
            lax.fori_loop(0, NG, agroup, 0)

            bin_totals_cum(SELB)
            d1 = find_cut(SELB, KTOP)
            cm1 = tot[pl.ds(jnp.maximum(d1 - 1, 0), 16)][0]
            smem[0] = d1
            smem[1] = 22
            smem[2] = jnp.where(d1 > 0, cm1, 0)
            smem[3] = tot[pl.ds(d1, 16)][0]

            for (s_prev, s_cur, wbits) in ((22, 12, 10), (12, 2, 10), (2, 0, 2)):
                nb = 1 << wbits
                nbt = max(nb, 16)

                @pl.when(smem[3] > CMAX)
                def _():
                    p_prev = lax.bitcast_convert_type(
                        jnp.full((16,), smem[0], jnp.int32), jnp.uint32)
                    cl0 = smem[2]
                    zero_hist(nbt * 16)

                    def rgroup(o, _c):
                        pvec = pbuf[pl.ds(o * 16, 16)]
                        for l in range(16):
                            g = gbuf[pl.ds(o * 256 + l * 16, 16)]
                            key = _mkkey(g, pvec[l])
                            mk = (key >> s_prev) == p_prev
                            d = lax.bitcast_convert_type(
                                (key >> s_cur) & jnp.uint32(nb - 1), jnp.int32)
                            plsc.addupdate_scatter(hist, [lsel + d], one16,
                                                   mask=mk)
                        return 0
                    lax.fori_loop(0, NG, rgroup, 0)

                    bin_totals_cum(nbt)
                    d2 = find_cut(nbt, KTOP - cl0)
                    cm2 = tot[pl.ds(jnp.maximum(d2 - 1, 0), 16)][0]
                    smem[3] = cl0 + tot[pl.ds(d2, 16)][0]
                    smem[2] = cl0 + jnp.where(d2 > 0, cm2, 0)
                    smem[0] = smem[0] * nb + d2
                    smem[1] = s_cur

            sfin = jnp.full((16,), smem[1], jnp.int32).astype(jnp.uint32)
            pfin = lax.bitcast_convert_type(
                jnp.full((16,), smem[0], jnp.int32), jnp.uint32)

            def egroup(o, off):
                pvec = pbuf[pl.ds(o * 16, 16)]
                for q4 in range(4):
                    ls_ = [o * 256 + (q4 * 4 + u) * 16 for u in range(4)]
                    gs = [gbuf[pl.ds(a, 16)] for a in ls_]
                    ks = [_mkkey(gs[u], pvec[q4 * 4 + u]) for u in range(4)]
                    mks = [(k >> sfin) <= pfin for k in ks]
                    kis = [lax.bitcast_convert_type(k, jnp.int32) for k in ks]
                    pcs = [plsc.all_reduce_population_count(m)[0] for m in mks]
                    for u in range(4):
                        offc = jnp.minimum(off, CMAX)
                        plsc.store_compressed(cka.at[pl.ds(offc, 16)],
                                              kis[u], mask=mks[u])
                        plsc.store_compressed(cia.at[pl.ds(offc, 16)],
                                              (o * 256 + (q4 * 4 + u) * 16)
                                              + lane, mask=mks[u])
                        off = off + pcs[u]
                return off
            off = lax.fori_loop(0, NG, egroup, jnp.int32(0))
            m_cnt = jnp.minimum(off, CMAX)
            m_pad = (m_cnt + 63) & ~63

            for pi in range(4):
                iv = m_cnt + pi * 16 + lane
                plsc.store_scatter(cka, [iv], ones16, mask=iv < m_pad)

            q = m_pad // 16
            qq = m_pad // 64
            bidx = lane * q
            for p in range(4):
                s_k, s_i, d_k, d_i = ((cka, cia, ckb, cib) if p % 2 == 0
                                      else (ckb, cib, cka, cia))
                sh = 8 * p

                def zs(i, _c):
                    for u in range(8):
                        hist[pl.ds((i * 8 + u) * 16, 16)] = zvec
                    return 0
                lax.fori_loop(0, SORTB // 8, zs, 0)

                def hpass(i, _c, s_k=s_k, sh=sh):
                    ks = [plsc.load_gather(s_k, [bidx + (i * 4 + u)])
                          for u in range(4)]
                    hs = [(lax.shift_right_logical(k, sh) & 0xFF) * 16 + lane
                          for k in ks]
                    for u in range(4):
                        plsc.addupdate_scatter(hist, [hs[u]], one16)
                    return 0
                lax.fori_loop(0, qq, hpass, 0)

                def scan(i, carry):
                    vs = [hist[pl.ds((i * 4 + u) * 16, 16)] for u in range(4)]
                    css = [plsc.cumsum(v) for v in vs]
                    for u in range(4):
                        hist[pl.ds((i * 4 + u) * 16, 16)] = (
                            css[u] - vs[u] + carry)
                        carry = carry + css[u][15]
                    return carry
                lax.fori_loop(0, SORTB // 4, scan, jnp.int32(0))

                def ppass(i, _c, s_k=s_k, s_i=s_i, d_k=d_k, d_i=d_i, sh=sh):
                    iis = [bidx + (i * 4 + u) for u in range(4)]
                    ks = [plsc.load_gather(s_k, [ii]) for ii in iis]
                    vs = [plsc.load_gather(s_i, [ii]) for ii in iis]
                    hs = [(lax.shift_right_logical(k, sh) & 0xFF) * 16 + lane
                          for k in ks]
                    for u in range(4):
                        r = plsc.load_gather(hist, [hs[u]])
                        plsc.store_scatter(hist, [hs[u]], r + 1)
                        plsc.store_scatter(d_k, [r], ks[u])
                        plsc.store_scatter(d_i, [r], vs[u])
                    return 0
                lax.fori_loop(0, qq, ppass, 0)

            @pl.when(pair == 0)
            def _():
                pltpu.sync_copy(cia.at[pl.ds(0, KTOP)], os_hbm.at[b])

            @pl.when(pair == 1)
            def _():
                pltpu.sync_copy(cia.at[pl.ds(0, KTOP)], ot_hbm.at[b])

        issue(0, gbuf0, pbuf0, sg0, sp0)

        def pair_of_tasks(jj, _c):
            j0 = jj * 2
            issue(j0 + 1, gbuf1, pbuf1, sg1, sp1)
            task(j0, gbuf0, pbuf0, sg0, sp0)

            @pl.when(jj < NROW // 2 - 1)
            def _():
                issue(j0 + 2, gbuf0, pbuf0, sg0, sp0)
            task(j0 + 1, gbuf1, pbuf1, sg1, sp1)
            return 0
        lax.fori_loop(0, NROW // 2, pair_of_tasks, 0)

    process()


def kernel(U_t, U_h, G_src, G_tgt, B):
    b = U_t.shape[0]
    ls, lt = _log_priors(U_t, U_h)

    mesh = plsc.VectorSubcoreMesh(core_axis_name="c", subcore_axis_name="s")
    sc = pl.kernel(
        _sc_body,
        mesh=mesh,
        compiler_params=pltpu.CompilerParams(needs_layout_passes=False),
        out_type=(jax.ShapeDtypeStruct((b, KTOP), jnp.int32),
                  jax.ShapeDtypeStruct((b, KTOP), jnp.int32)),
        scratch_types=[
            pltpu.VMEM((NTOK,), jnp.float32),
            pltpu.VMEM((NTOK,), jnp.float32),
            pltpu.VMEM((TH + 16,), jnp.float32),
            pltpu.VMEM((TH + 16,), jnp.float32),
            pltpu.VMEM((SELB * 16,), jnp.int32),
            pltpu.VMEM((SELB + 16,), jnp.int32),
            pltpu.VMEM((CBUF,), jnp.int32),
            pltpu.VMEM((CBUF,), jnp.int32),
            pltpu.VMEM((CBUF,), jnp.int32),
            pltpu.VMEM((CBUF,), jnp.int32),
            pltpu.SMEM((8,), jnp.int32),
            pltpu.SemaphoreType.DMA,
            pltpu.SemaphoreType.DMA,
            pltpu.SemaphoreType.DMA,
            pltpu.SemaphoreType.DMA,
        ],
    )
    src_idx, tgt_idx = sc(G_src, G_tgt, ls, lt)
    return (src_idx, tgt_idx)


# speedup vs baseline: 3.1732x; 1.1348x over previous
import jax
import jax.numpy as jnp
from jax import lax
from jax.experimental import pallas as pl
from jax.experimental.pallas import tpu as pltpu
from jax.experimental.pallas import tpu_sc as plsc

EPS = 1e-4
NTOK = 32768
TH = 2048
KTOP = 2048
NG = TH // 16
CMAX = 8192
CLIM = CMAX + 16
CBUF = CMAX + 80
SELB = 1024
SORTB = 256
NROW = 64


def _log_priors(U_t, U_h):
    b = U_t.shape[0]
    t, h = U_t.shape[1], U_h.shape[1]
    ut_e = jnp.broadcast_to(U_t[:, :, None], (b, t, h)).reshape(b, TH)
    uh_e = jnp.broadcast_to(U_h[:, None, :], (b, t, h)).reshape(b, TH)
    fs = jnp.exp(jnp.log(ut_e) / 1.0) * jnp.exp(jnp.log(uh_e) / 2.0)
    fs = jnp.clip(fs, EPS, 1.0 - EPS)
    ft = jnp.exp(jnp.log(1.0 - ut_e) / 1.0) * jnp.exp(jnp.log(1.0 - uh_e) / 2.0)
    ft = jnp.clip(ft, EPS, 1.0 - EPS)
    return jnp.log(fs), jnp.log(ft)


def _mkkey(g, pv):
    s = g + pv
    bits = lax.bitcast_convert_type(s, jnp.uint32)
    neg = bits >= jnp.uint32(0x80000000)
    m = jnp.where(neg, ~bits, bits | jnp.uint32(0x80000000))
    return ~m


def _sc_body(gs_hbm, gt_hbm, ps_hbm, pt_hbm, os_hbm, ot_hbm,
             gbuf0, gbuf1, pbuf0, pbuf1, hist, tot, cka, cia, ckb, cib,
             smem, sg0, sg1, sp0, sp1):
    cid = lax.axis_index("c")
    sid = lax.axis_index("s")
    wid = sid * 2 + cid
    pair = wid % 2
    base = wid // 2
    lane = lax.iota(jnp.int32, 16)
    one16 = jnp.ones((16,), jnp.int32)
    ones16 = jnp.full((16,), -1, jnp.int32)
    lsel = lane * SELB

    zvec = jnp.zeros((16,), jnp.int32)

    def zero_hist(nwords):
        def z(i, _):
            for u in range(8):
                hist[pl.ds((i * 8 + u) * 16, 16)] = zvec
            return 0
        lax.fori_loop(0, nwords // 128, z, 0)

    def bin_totals_cum(nbins):
        def tots(t_, carry):
            acc = hist[pl.ds(t_ * 16, 16)]
            for l in range(1, 16):
                acc = acc + hist[pl.ds(l * SELB + t_ * 16, 16)]
            cst = plsc.cumsum(acc) + carry
            tot[pl.ds(t_ * 16, 16)] = cst
            return jnp.max(cst)
        lax.fori_loop(0, nbins // 16, tots, jnp.int32(0))

    def find_cut(nbins, kneed):
        def findd(t_, acc):
            v = tot[pl.ds(t_ * 16, 16)]
            return acc + jnp.sum(jnp.where(v < kneed, 1, 0))
        return lax.fori_loop(0, nbins // 16, findd, jnp.int32(0))

    def process():
        def issue(j, gbuf, pbuf, sg, sp):
            b = base + 16 * j

            @pl.when(pair == 0)
            def _():
                pltpu.make_async_copy(gs_hbm.at[b], gbuf, sg).start()
                pltpu.make_async_copy(ps_hbm.at[b], pbuf.at[pl.ds(0, TH)],
                                      sp).start()

            @pl.when(pair == 1)
            def _():
                pltpu.make_async_copy(gt_hbm.at[b], gbuf, sg).start()
                pltpu.make_async_copy(pt_hbm.at[b], pbuf.at[pl.ds(0, TH)],
                                      sp).start()

        def task(j, gbuf, pbuf, sg, sp):
            b = base + 16 * j
            pltpu.make_async_copy(gs_hbm.at[b], gbuf, sg).wait()
            pltpu.make_async_copy(ps_hbm.at[b], pbuf.at[pl.ds(0, TH)],
                                  sp).wait()

            zero_hist(SELB * 16)

            def agroup(o, _c):
                pvec = pbuf[pl.ds(o * 16, 16)]
                for q8 in range(2):
                    ls_ = [o * 256 + (q8 * 8 + u) * 16 for u in range(8)]
                    gs = [gbuf[pl.ds(a, 16)] for a in ls_]
                    ks = [_mkkey(gs[u], pvec[q8 * 8 + u]) for u in range(8)]
                    ds_ = [lax.bitcast_convert_type(k >> 22, jnp.int32)
                           for k in ks]
                    for u in range(8):
                        plsc.addupdate_scatter(hist, [lsel + ds_[u]], one16)
                return 0
            lax.fori_loop(0, NG, agroup, 0)

            bin_totals_cum(SELB)
            d1 = find_cut(SELB, KTOP)
            cm1 = tot[pl.ds(jnp.maximum(d1 - 1, 0), 16)][0]
            smem[0] = d1
            smem[1] = 22
            smem[2] = jnp.where(d1 > 0, cm1, 0)
            smem[3] = tot[pl.ds(d1, 16)][0]

            for (s_prev, s_cur, wbits) in ((22, 12, 10), (12, 2, 10), (2, 0, 2)):
                nb = 1 << wbits
                nbt = max(nb, 16)

                @pl.when(smem[3] > CMAX)
                def _():
                    p_prev = lax.bitcast_convert_type(
                        jnp.full((16,), smem[0], jnp.int32), jnp.uint32)
                    cl0 = smem[2]
                    zero_hist(nbt * 16)

                    def rgroup(o, _c):
                        pvec = pbuf[pl.ds(o * 16, 16)]
                        for l in range(16):
                            g = gbuf[pl.ds(o * 256 + l * 16, 16)]
                            key = _mkkey(g, pvec[l])
                            mk = (key >> s_prev) == p_prev
                            d = lax.bitcast_convert_type(
                                (key >> s_cur) & jnp.uint32(nb - 1), jnp.int32)
                            plsc.addupdate_scatter(hist, [lsel + d], one16,
                                                   mask=mk)
                        return 0
                    lax.fori_loop(0, NG, rgroup, 0)

                    bin_totals_cum(nbt)
                    d2 = find_cut(nbt, KTOP - cl0)
                    cm2 = tot[pl.ds(jnp.maximum(d2 - 1, 0), 16)][0]
                    smem[3] = cl0 + tot[pl.ds(d2, 16)][0]
                    smem[2] = cl0 + jnp.where(d2 > 0, cm2, 0)
                    smem[0] = smem[0] * nb + d2
                    smem[1] = s_cur

            sfin = jnp.full((16,), smem[1], jnp.int32).astype(jnp.uint32)
            pfin = lax.bitcast_convert_type(
                jnp.full((16,), smem[0], jnp.int32), jnp.uint32)

            def egroup(o, off):
                pvec = pbuf[pl.ds(o * 16, 16)]
                for q8 in range(2):
                    ls_ = [o * 256 + (q8 * 8 + u) * 16 for u in range(8)]
                    gs = [gbuf[pl.ds(a, 16)] for a in ls_]
                    ks = [_mkkey(gs[u], pvec[q8 * 8 + u]) for u in range(8)]
                    mks = [(k >> sfin) <= pfin for k in ks]
                    kis = [lax.bitcast_convert_type(k, jnp.int32) for k in ks]
                    pcs = [plsc.all_reduce_population_count(m)[0] for m in mks]
                    for u in range(8):
                        offc = jnp.minimum(off, CMAX)
                        plsc.store_compressed(cka.at[pl.ds(offc, 16)],
                                              kis[u], mask=mks[u])
                        plsc.store_compressed(cia.at[pl.ds(offc, 16)],
                                              ls_[u] + lane, mask=mks[u])
                        off = off + pcs[u]
                return off
            off = lax.fori_loop(0, NG, egroup, jnp.int32(0))
            m_cnt = jnp.minimum(off, CMAX)
            m_pad = (m_cnt + 63) & ~63

            for pi in range(4):
                iv = m_cnt + pi * 16 + lane
                plsc.store_scatter(cka, [iv], ones16, mask=iv < m_pad)

            q = m_pad // 16
            qq = m_pad // 64
            bidx = lane * q
            for p in range(4):
                s_k, s_i, d_k, d_i = ((cka, cia, ckb, cib) if p % 2 == 0
                                      else (ckb, cib, cka, cia))
                sh = 8 * p

                def zs(i, _c):
                    for u in range(8):
                        hist[pl.ds((i * 8 + u) * 16, 16)] = zvec
                    return 0
                lax.fori_loop(0, SORTB // 8, zs, 0)

                def hpass(i, _c, s_k=s_k, sh=sh):
                    ks = [plsc.load_gather(s_k, [bidx + (i * 4 + u)])
                          for u in range(4)]
                    hs = [(lax.shift_right_logical(k, sh) & 0xFF) * 16 + lane
                          for k in ks]
                    for u in range(4):
                        plsc.addupdate_scatter(hist, [hs[u]], one16)
                    return 0
                lax.fori_loop(0, qq, hpass, 0)

                def scan(i, carry):
                    vs = [hist[pl.ds((i * 4 + u) * 16, 16)] for u in range(4)]
                    css = [plsc.cumsum(v) for v in vs]
                    for u in range(4):
                        hist[pl.ds((i * 4 + u) * 16, 16)] = (
                            css[u] - vs[u] + carry)
                        carry = carry + css[u][15]
                    return carry
                lax.fori_loop(0, SORTB // 4, scan, jnp.int32(0))

                def ppass(i, _c, s_k=s_k, s_i=s_i, d_k=d_k, d_i=d_i, sh=sh):
                    iis = [bidx + (i * 4 + u) for u in range(4)]
                    ks = [plsc.load_gather(s_k, [ii]) for ii in iis]
                    vs = [plsc.load_gather(s_i, [ii]) for ii in iis]
                    hs = [(lax.shift_right_logical(k, sh) & 0xFF) * 16 + lane
                          for k in ks]
                    for u in range(4):
                        r = plsc.load_gather(hist, [hs[u]])
                        plsc.store_scatter(hist, [hs[u]], r + 1)
                        plsc.store_scatter(d_k, [r], ks[u])
                        plsc.store_scatter(d_i, [r], vs[u])
                    return 0
                lax.fori_loop(0, qq, ppass, 0)

            @pl.when(pair == 0)
            def _():
                pltpu.sync_copy(cia.at[pl.ds(0, KTOP)], os_hbm.at[b])

            @pl.when(pair == 1)
            def _():
                pltpu.sync_copy(cia.at[pl.ds(0, KTOP)], ot_hbm.at[b])

        issue(0, gbuf0, pbuf0, sg0, sp0)

        def pair_of_tasks(jj, _c):
            j0 = jj * 2
            issue(j0 + 1, gbuf1, pbuf1, sg1, sp1)
            task(j0, gbuf0, pbuf0, sg0, sp0)

            @pl.when(jj < NROW // 2 - 1)
            def _():
                issue(j0 + 2, gbuf0, pbuf0, sg0, sp0)
            task(j0 + 1, gbuf1, pbuf1, sg1, sp1)
            return 0
        lax.fori_loop(0, NROW // 2, pair_of_tasks, 0)

    process()


def kernel(U_t, U_h, G_src, G_tgt, B):
    b = U_t.shape[0]
    ls, lt = _log_priors(U_t, U_h)

    mesh = plsc.VectorSubcoreMesh(core_axis_name="c", subcore_axis_name="s")
    sc = pl.kernel(
        _sc_body,
        mesh=mesh,
        compiler_params=pltpu.CompilerParams(needs_layout_passes=False),
        out_type=(jax.ShapeDtypeStruct((b, KTOP), jnp.int32),
                  jax.ShapeDtypeStruct((b, KTOP), jnp.int32)),
        scratch_types=[
            pltpu.VMEM((NTOK,), jnp.float32),
            pltpu.VMEM((NTOK,), jnp.float32),
            pltpu.VMEM((TH + 16,), jnp.float32),
            pltpu.VMEM((TH + 16,), jnp.float32),
            pltpu.VMEM((SELB * 16,), jnp.int32),
            pltpu.VMEM((SELB + 16,), jnp.int32),
            pltpu.VMEM((CBUF,), jnp.int32),
            pltpu.VMEM((CBUF,), jnp.int32),
            pltpu.VMEM((CBUF,), jnp.int32),
            pltpu.VMEM((CBUF,), jnp.int32),
            pltpu.SMEM((8,), jnp.int32),
            pltpu.SemaphoreType.DMA,
            pltpu.SemaphoreType.DMA,
            pltpu.SemaphoreType.DMA,
            pltpu.SemaphoreType.DMA,
        ],
    )
    src_idx, tgt_idx = sc(G_src, G_tgt, ls, lt)
    return (src_idx, tgt_idx)
